# baseline (speedup 1.0000x reference)
import functools

import jax
import jax.numpy as jnp
import numpy as np
from jax import lax
from jax.experimental import pallas as pl
from jax.experimental.pallas import tpu as pltpu
from jax.experimental.pallas import tpu_sc as plsc

NUM_NODES = 100000
K = 64
D = 128
N = 4096
NW = 32
TPW = N // NW
TOK_PER_CHUNK = 2
CHUNK_SLOTS = TOK_PER_CHUNK * K
NCHUNK = TPW // TOK_PER_CHUNK
NBUF = 2
HOT = 4096


def _threefry2x32(k0, k1, x0, x1):
    def rotl(v, r):
        return ((v << np.uint32(r)) | (v >> np.uint32(32 - r))).astype(np.uint32)

    rots = ((13, 15, 26, 6), (17, 29, 16, 24))
    ks = (k0, k1, np.uint32(k0 ^ k1 ^ np.uint32(0x1BD11BDA)))
    x0 = (x0 + ks[0]).astype(np.uint32)
    x1 = (x1 + ks[1]).astype(np.uint32)
    for i in range(5):
        for r in rots[i % 2]:
            x0 = (x0 + x1).astype(np.uint32)
            x1 = np.uint32(rotl(x1, r) ^ x0)
        x0 = (x0 + ks[(i + 1) % 3]).astype(np.uint32)
        x1 = (x1 + ks[(i + 2) % 3] + np.uint32(i + 1)).astype(np.uint32)
    return x0, x1


def _uniform_bits(seed, num):
    k0 = np.uint32(np.uint64(seed) >> np.uint64(32))
    k1 = np.uint32(np.uint64(seed) & np.uint64(0xFFFFFFFF))
    x0, x1 = _threefry2x32(k0, k1, np.zeros(num, np.uint32),
                           np.arange(num, dtype=np.uint32))
    bits = x0 ^ x1
    f = ((bits >> np.uint32(9)) | np.uint32(0x3F800000)).view(np.float32)
    return f - np.float32(1.0)


def _draw_neg_indices():
    k = np.arange(NUM_NODES, dtype=np.float32)
    w = (np.log(k + np.float32(2.0)) - np.log(k + np.float32(1.0))).astype(
        np.float32) / np.float32(np.log(np.float32(NUM_NODES + 1)))
    w = (w / np.float32(np.sqrt(np.sum(w * w, dtype=np.float32)))).astype(
        np.float32)
    cdf = np.cumsum(
        (w / np.float32(np.sum(w, dtype=np.float32))).astype(np.float32),
        dtype=np.float32)
    u = _uniform_bits(12345, K * N)
    idx = np.clip(np.searchsorted(cdf, u), 0, NUM_NODES - 1).astype(np.int32)
    return np.sort(idx.reshape(N, K), axis=1)


def _build_plan():
    negs = _draw_neg_indices()
    hcnt = (negs < HOT).sum(axis=1)

    cold_per_chunk = np.zeros(NW * NCHUNK, np.int32)
    for c in range(NW * NCHUNK):
        t0, t1 = 2 * c, 2 * c + 1
        cold_per_chunk[c] = (K - hcnt[t0]) + (K - hcnt[t1])
    cmax = int(((cold_per_chunk.max() + 7) // 8) * 8)

    hot_list = np.zeros((NW, NCHUNK, CHUNK_SLOTS), np.int32)
    cold_list = np.zeros((NW, NCHUNK, cmax), np.int32)
    rloc = np.zeros((NW, NCHUNK, CHUNK_SLOTS), np.int32)
    for w in range(NW):
        for c in range(NCHUNK):
            g = w * NCHUNK + c
            t0, t1 = 2 * g, 2 * g + 1
            h0, h1 = int(hcnt[t0]), int(hcnt[t1])
            c0, c1 = K - h0, K - h1
            hl = np.full(CHUNK_SLOTS, g % HOT, np.int32)
            hl[:h0] = negs[t0, :h0]
            hl[h0:h0 + h1] = negs[t1, :h1]
            hot_list[w, c] = hl
            cl = np.full(cmax, HOT + (g * 131) % (NUM_NODES - HOT), np.int32)
            cl[:c0] = negs[t0, h0:]
            cl[c0:c0 + c1] = negs[t1, h1:]
            cold_list[w, c] = cl
            r = np.empty(CHUNK_SLOTS, np.int32)
            r[:h0] = np.arange(h0)
            r[h0:K] = CHUNK_SLOTS + np.arange(c0)
            r[K:K + h1] = h0 + np.arange(h1)
            r[K + h1:] = CHUNK_SLOTS + c0 + np.arange(c1)
            rloc[w, c] = r
    return hot_list, cold_list, rloc, cmax


_HOT_LIST, _COLD_LIST, _RLOC, CMAX = _build_plan()
ROWS_BUF = CHUNK_SLOTS + CMAX


def _sc_scores_body(emb_hbm, ctx_hbm, hotl_hbm, coldl_hbm, rloc_hbm, pos_hbm,
                    nout_hbm, pout_hbm,
                    emb_v, hotl_v, coldl_v, rloc_v, pos_v,
                    rows_a, rows_b, nsc_a, nsc_b, psc_v, hot_sh,
                    sem_a, sem_b, sem_ca, sem_cb, sem_oa, sem_ob, sem_p):
    cid = lax.axis_index("c")
    sid = lax.axis_index("s")
    wid = sid * 2 + cid
    base_n = wid * TPW

    span = HOT // 16
    pltpu.sync_copy(ctx_hbm.at[pl.ds(sid * span, span)],
                    hot_sh.at[pl.ds(sid * span, span)])

    pltpu.sync_copy(emb_hbm.at[pl.ds(base_n, TPW)], emb_v)
    pltpu.sync_copy(hotl_hbm.at[wid], hotl_v)
    pltpu.sync_copy(coldl_hbm.at[wid], coldl_v)
    pltpu.sync_copy(rloc_hbm.at[wid], rloc_v)
    pltpu.sync_copy(pos_hbm.at[wid], pos_v)

    plsc.subcore_barrier()

    rows = [rows_a, rows_b]
    sems = [sem_a, sem_b]
    csems = [sem_ca, sem_cb]
    nscs = [nsc_a, nsc_b]
    osems = [sem_oa, sem_ob]

    def fire(c, b):
        pltpu.async_copy(hot_sh.at[hotl_v.at[c]],
                         rows[b].at[pl.ds(0, CHUNK_SLOTS)], sems[b])
        pltpu.async_copy(ctx_hbm.at[coldl_v.at[c]],
                         rows[b].at[pl.ds(CHUNK_SLOTS, CMAX)], csems[b])

    def drain(b):
        pltpu.make_async_copy(ctx_hbm.at[pl.ds(0, CHUNK_SLOTS)],
                              rows[b].at[pl.ds(0, CHUNK_SLOTS)],
                              sems[b]).wait()
        pltpu.make_async_copy(ctx_hbm.at[pl.ds(0, CMAX)],
                              rows[b].at[pl.ds(CHUNK_SLOTS, CMAX)],
                              csems[b]).wait()

    for b in range(NBUF):
        fire(b, b)

    def compute_chunk(rows_ref, nsc_ref, c):
        for t in range(TOK_PER_CHUNK):
            for kb in range(0, K // 16, 2):
                rvec0 = rloc_v[c, pl.ds(t * K + kb * 16, 16)]
                rvec1 = rloc_v[c, pl.ds(t * K + (kb + 1) * 16, 16)]

                def dbody(i, carry):
                    a0, a1, dvec = carry
                    evec = emb_v[c * TOK_PER_CHUNK + t, pl.ds(i * 16, 16)]
                    for u in range(16):
                        e = evec[u]
                        v0 = plsc.load_gather(rows_ref, [rvec0, dvec])
                        v1 = plsc.load_gather(rows_ref, [rvec1, dvec])
                        a0 = a0 + v0 * e
                        a1 = a1 + v1 * e
                        dvec = dvec + 1
                    return a0, a1, dvec

                a0, a1, _ = lax.fori_loop(
                    0, D // 16, dbody,
                    (jnp.zeros((16,), jnp.float32),
                     jnp.zeros((16,), jnp.float32),
                     jnp.zeros((16,), jnp.int32)))
                nsc_ref[t, pl.ds(kb * 16, 16)] = a0
                nsc_ref[t, pl.ds((kb + 1) * 16, 16)] = a1

    def body(i, _):
        c0 = NBUF * i
        for b in range(NBUF):
            drain(b)

            @pl.when(i > 0)
            def _():
                pltpu.make_async_copy(
                    ctx_hbm.at[pl.ds(0, TOK_PER_CHUNK)],
                    nout_hbm.at[pl.ds(0, TOK_PER_CHUNK)], osems[b]).wait()

            compute_chunk(rows[b], nscs[b], c0 + b)
            pltpu.async_copy(
                nscs[b],
                nout_hbm.at[pl.ds(base_n + (c0 + b) * TOK_PER_CHUNK,
                                  TOK_PER_CHUNK)],
                osems[b])

            @pl.when(i < NCHUNK // NBUF - 1)
            def _():
                fire(c0 + b + NBUF, b)

        return 0

    lax.fori_loop(0, NCHUNK // NBUF, body, 0)
    for b in range(NBUF):
        pltpu.make_async_copy(
            ctx_hbm.at[pl.ds(0, TOK_PER_CHUNK)],
            nout_hbm.at[pl.ds(0, TOK_PER_CHUNK)], osems[b]).wait()

    pltpu.async_copy(ctx_hbm.at[pos_v], rows_a.at[pl.ds(0, TPW)], sem_p)
    pltpu.make_async_copy(ctx_hbm.at[pl.ds(0, TPW)],
                          rows_a.at[pl.ds(0, TPW)], sem_p).wait()

    for tb in range(TPW // 16):
        tvec = tb * 16 + lax.iota(jnp.int32, 16)

        def pbody(i, carry):
            acc, dvec = carry
            for u in range(8):
                va = plsc.load_gather(rows_a, [tvec, dvec])
                vb = plsc.load_gather(emb_v, [tvec, dvec])
                acc = acc + va * vb
                dvec = dvec + 1
            return acc, dvec

        acc, _ = lax.fori_loop(
            0, D // 8, pbody,
            (jnp.zeros((16,), jnp.float32), jnp.zeros((16,), jnp.int32)))
        psc_v[pl.ds(tb * 16, 16)] = acc

    pltpu.sync_copy(psc_v, pout_hbm.at[pl.ds(base_n, TPW)])


@functools.cache
def _make_sc_scores():
    return pl.kernel(
        _sc_scores_body,
        mesh=plsc.VectorSubcoreMesh(core_axis_name="c", subcore_axis_name="s"),
        compiler_params=pltpu.CompilerParams(needs_layout_passes=False),
        out_type=(jax.ShapeDtypeStruct((N, K), jnp.float32),
                  jax.ShapeDtypeStruct((N,), jnp.float32)),
        scratch_types=[
            pltpu.VMEM((TPW, D), jnp.float32),
            pltpu.VMEM((NCHUNK, CHUNK_SLOTS), jnp.int32),
            pltpu.VMEM((NCHUNK, CMAX), jnp.int32),
            pltpu.VMEM((NCHUNK, CHUNK_SLOTS), jnp.int32),
            pltpu.VMEM((TPW,), jnp.int32),
            pltpu.VMEM((ROWS_BUF, D), jnp.float32),
            pltpu.VMEM((ROWS_BUF, D), jnp.float32),
            pltpu.VMEM((TOK_PER_CHUNK, K), jnp.float32),
            pltpu.VMEM((TOK_PER_CHUNK, K), jnp.float32),
            pltpu.VMEM((TPW,), jnp.float32),
            pltpu.VMEM_SHARED((HOT, D), jnp.float32),
            pltpu.SemaphoreType.DMA,
            pltpu.SemaphoreType.DMA,
            pltpu.SemaphoreType.DMA,
            pltpu.SemaphoreType.DMA,
            pltpu.SemaphoreType.DMA,
            pltpu.SemaphoreType.DMA,
            pltpu.SemaphoreType.DMA,
        ],
    )


def _tc_loss_body(nsc_ref, psc_ref, out_ref):
    ns = nsc_ref[...]
    ps = psc_ref[...]
    def logsig(x):
        return jnp.minimum(x, 0.0) - jnp.log1p(jnp.exp(-jnp.abs(x)))
    total = jnp.sum(logsig(-ns)) + jnp.sum(logsig(ps))
    out_ref[...] = jnp.reshape(-total / np.float32(N), (1, 1))


def kernel(embed, pos_neighbors, ctx_weight):
    pos = pos_neighbors.reshape(NW, TPW)
    nsc, psc = _make_sc_scores()(
        embed, ctx_weight, jnp.asarray(_HOT_LIST), jnp.asarray(_COLD_LIST),
        jnp.asarray(_RLOC), pos)
    loss = pl.pallas_call(
        _tc_loss_body,
        out_shape=jax.ShapeDtypeStruct((1, 1), jnp.float32),
    )(nsc, psc.reshape(N, 1))
    return loss.reshape(())


# speedup vs baseline: 1.2057x; 1.2057x over previous
import functools

import jax
import jax.numpy as jnp
import numpy as np
from jax import lax
from jax.experimental import pallas as pl
from jax.experimental.pallas import tpu as pltpu
from jax.experimental.pallas import tpu_sc as plsc

NUM_NODES = 100000
K = 64
D = 128
N = 4096
NW = 32
TPW = N // NW
TOK_PER_CHUNK = 2
CHUNK_SLOTS = TOK_PER_CHUNK * K
NCHUNK = TPW // TOK_PER_CHUNK
NBUF = 2
HOT = 4096


def _threefry2x32(k0, k1, x0, x1):
    def rotl(v, r):
        return ((v << np.uint32(r)) | (v >> np.uint32(32 - r))).astype(np.uint32)

    rots = ((13, 15, 26, 6), (17, 29, 16, 24))
    ks = (k0, k1, np.uint32(k0 ^ k1 ^ np.uint32(0x1BD11BDA)))
    x0 = (x0 + ks[0]).astype(np.uint32)
    x1 = (x1 + ks[1]).astype(np.uint32)
    for i in range(5):
        for r in rots[i % 2]:
            x0 = (x0 + x1).astype(np.uint32)
            x1 = np.uint32(rotl(x1, r) ^ x0)
        x0 = (x0 + ks[(i + 1) % 3]).astype(np.uint32)
        x1 = (x1 + ks[(i + 2) % 3] + np.uint32(i + 1)).astype(np.uint32)
    return x0, x1


def _uniform_bits(seed, num):
    k0 = np.uint32(np.uint64(seed) >> np.uint64(32))
    k1 = np.uint32(np.uint64(seed) & np.uint64(0xFFFFFFFF))
    x0, x1 = _threefry2x32(k0, k1, np.zeros(num, np.uint32),
                           np.arange(num, dtype=np.uint32))
    bits = x0 ^ x1
    f = ((bits >> np.uint32(9)) | np.uint32(0x3F800000)).view(np.float32)
    return f - np.float32(1.0)


def _draw_neg_indices():
    k = np.arange(NUM_NODES, dtype=np.float32)
    w = (np.log(k + np.float32(2.0)) - np.log(k + np.float32(1.0))).astype(
        np.float32) / np.float32(np.log(np.float32(NUM_NODES + 1)))
    w = (w / np.float32(np.sqrt(np.sum(w * w, dtype=np.float32)))).astype(
        np.float32)
    cdf = np.cumsum(
        (w / np.float32(np.sum(w, dtype=np.float32))).astype(np.float32),
        dtype=np.float32)
    u = _uniform_bits(12345, K * N)
    idx = np.clip(np.searchsorted(cdf, u), 0, NUM_NODES - 1).astype(np.int32)
    return np.sort(idx.reshape(N, K), axis=1)


def _build_plan():
    negs = _draw_neg_indices()
    hcnt = (negs < HOT).sum(axis=1)

    cold_per_chunk = np.zeros(NW * NCHUNK, np.int32)
    for c in range(NW * NCHUNK):
        t0, t1 = 2 * c, 2 * c + 1
        cold_per_chunk[c] = (K - hcnt[t0]) + (K - hcnt[t1])
    cmax = int(((cold_per_chunk.max() + 7) // 8) * 8)

    hot_list = np.zeros((NW, NCHUNK, CHUNK_SLOTS), np.int32)
    cold_list = np.zeros((NW, NCHUNK, cmax), np.int32)
    rloc = np.zeros((NW, NCHUNK, CHUNK_SLOTS), np.int32)
    for w in range(NW):
        for c in range(NCHUNK):
            g = w * NCHUNK + c
            t0, t1 = 2 * g, 2 * g + 1
            h0, h1 = int(hcnt[t0]), int(hcnt[t1])
            c0, c1 = K - h0, K - h1
            hl = np.full(CHUNK_SLOTS, g % HOT, np.int32)
            hl[:h0] = negs[t0, :h0]
            hl[h0:h0 + h1] = negs[t1, :h1]
            hot_list[w, c] = hl
            cl = np.full(cmax, HOT + (g * 131) % (NUM_NODES - HOT), np.int32)
            cl[:c0] = negs[t0, h0:]
            cl[c0:c0 + c1] = negs[t1, h1:]
            cold_list[w, c] = cl
            r = np.empty(CHUNK_SLOTS, np.int32)
            r[:h0] = np.arange(h0)
            r[h0:K] = CHUNK_SLOTS + np.arange(c0)
            r[K:K + h1] = h0 + np.arange(h1)
            r[K + h1:] = CHUNK_SLOTS + c0 + np.arange(c1)
            rloc[w, c] = r
    return hot_list, cold_list, rloc, cmax


_HOT_LIST, _COLD_LIST, _RLOC, CMAX = _build_plan()
ROWS_BUF = CHUNK_SLOTS + CMAX


def _sc_scores_body(emb_hbm, ctx_hbm, hotl_hbm, coldl_hbm, rloc_hbm, pos_hbm,
                    nout_hbm, pout_hbm,
                    emb_v, hotl_v, coldl_v, rloc_v, pos_v,
                    rows_a, rows_b, nsc_a, nsc_b, psc_v, hot_sh,
                    sem_a, sem_b, sem_ca, sem_cb, sem_oa, sem_ob, sem_p):
    cid = lax.axis_index("c")
    sid = lax.axis_index("s")
    wid = sid * 2 + cid
    base_n = wid * TPW

    span = HOT // 16
    pltpu.sync_copy(ctx_hbm.at[pl.ds(sid * span, span)],
                    hot_sh.at[pl.ds(sid * span, span)])

    pltpu.sync_copy(emb_hbm.at[pl.ds(base_n, TPW)], emb_v)
    pltpu.sync_copy(hotl_hbm.at[wid], hotl_v)
    pltpu.sync_copy(coldl_hbm.at[wid], coldl_v)
    pltpu.sync_copy(rloc_hbm.at[wid], rloc_v)
    pltpu.sync_copy(pos_hbm.at[wid], pos_v)

    plsc.subcore_barrier()

    rows = [rows_a, rows_b]
    sems = [sem_a, sem_b]
    csems = [sem_ca, sem_cb]
    nscs = [nsc_a, nsc_b]
    osems = [sem_oa, sem_ob]

    def fire(c, b):
        pltpu.async_copy(hot_sh.at[hotl_v.at[c]],
                         rows[b].at[pl.ds(0, CHUNK_SLOTS)], sems[b])
        pltpu.async_copy(ctx_hbm.at[coldl_v.at[c]],
                         rows[b].at[pl.ds(CHUNK_SLOTS, CMAX)], csems[b])

    def drain(b):
        pltpu.make_async_copy(ctx_hbm.at[pl.ds(0, CHUNK_SLOTS)],
                              rows[b].at[pl.ds(0, CHUNK_SLOTS)],
                              sems[b]).wait()
        pltpu.make_async_copy(ctx_hbm.at[pl.ds(0, CMAX)],
                              rows[b].at[pl.ds(CHUNK_SLOTS, CMAX)],
                              csems[b]).wait()

    for b in range(NBUF):
        fire(b, b)

    def compute_chunk(rows_ref, nsc_ref, c):
        for t in range(TOK_PER_CHUNK):
            for kb in range(K // 16):
                rvec = rloc_v[c, pl.ds((t * K + kb * 16), 16)]

                def dbody(i, carry):
                    acc, dvec = carry
                    evec = emb_v[c * TOK_PER_CHUNK + t, pl.ds(i * 16, 16)]
                    for u in range(16):
                        v = plsc.load_gather(rows_ref, [rvec, dvec])
                        acc = acc + v * evec[u]
                        dvec = dvec + 1
                    return acc, dvec

                acc, _ = lax.fori_loop(
                    0, D // 16, dbody,
                    (jnp.zeros((16,), jnp.float32),
                     jnp.zeros((16,), jnp.int32)))
                nsc_ref[t, pl.ds(kb * 16, 16)] = acc

    def body(i, _):
        c0 = NBUF * i
        for b in range(NBUF):
            drain(b)

            @pl.when(i > 0)
            def _():
                pltpu.make_async_copy(
                    ctx_hbm.at[pl.ds(0, TOK_PER_CHUNK)],
                    nout_hbm.at[pl.ds(0, TOK_PER_CHUNK)], osems[b]).wait()

            compute_chunk(rows[b], nscs[b], c0 + b)
            pltpu.async_copy(
                nscs[b],
                nout_hbm.at[pl.ds(base_n + (c0 + b) * TOK_PER_CHUNK,
                                  TOK_PER_CHUNK)],
                osems[b])

            @pl.when(i < NCHUNK // NBUF - 1)
            def _():
                fire(c0 + b + NBUF, b)

        return 0

    lax.fori_loop(0, NCHUNK // NBUF, body, 0)
    for b in range(NBUF):
        pltpu.make_async_copy(
            ctx_hbm.at[pl.ds(0, TOK_PER_CHUNK)],
            nout_hbm.at[pl.ds(0, TOK_PER_CHUNK)], osems[b]).wait()

    pltpu.async_copy(ctx_hbm.at[pos_v], rows_a.at[pl.ds(0, TPW)], sem_p)
    pltpu.make_async_copy(ctx_hbm.at[pl.ds(0, TPW)],
                          rows_a.at[pl.ds(0, TPW)], sem_p).wait()

    for tb in range(TPW // 16):
        tvec = tb * 16 + lax.iota(jnp.int32, 16)

        def pbody(i, carry):
            acc, dvec = carry
            for u in range(8):
                va = plsc.load_gather(rows_a, [tvec, dvec])
                vb = plsc.load_gather(emb_v, [tvec, dvec])
                acc = acc + va * vb
                dvec = dvec + 1
            return acc, dvec

        acc, _ = lax.fori_loop(
            0, D // 8, pbody,
            (jnp.zeros((16,), jnp.float32), jnp.zeros((16,), jnp.int32)))
        psc_v[pl.ds(tb * 16, 16)] = acc

    pltpu.sync_copy(psc_v, pout_hbm.at[pl.ds(base_n, TPW)])


@functools.cache
def _make_sc_scores():
    return pl.kernel(
        _sc_scores_body,
        mesh=plsc.VectorSubcoreMesh(core_axis_name="c", subcore_axis_name="s"),
        compiler_params=pltpu.CompilerParams(needs_layout_passes=False),
        out_type=(jax.ShapeDtypeStruct((N, K), jnp.float32),
                  jax.ShapeDtypeStruct((N,), jnp.float32)),
        scratch_types=[
            pltpu.VMEM((TPW, D), jnp.float32),
            pltpu.VMEM((NCHUNK, CHUNK_SLOTS), jnp.int32),
            pltpu.VMEM((NCHUNK, CMAX), jnp.int32),
            pltpu.VMEM((NCHUNK, CHUNK_SLOTS), jnp.int32),
            pltpu.VMEM((TPW,), jnp.int32),
            pltpu.VMEM((ROWS_BUF, D), jnp.float32),
            pltpu.VMEM((ROWS_BUF, D), jnp.float32),
            pltpu.VMEM((TOK_PER_CHUNK, K), jnp.float32),
            pltpu.VMEM((TOK_PER_CHUNK, K), jnp.float32),
            pltpu.VMEM((TPW,), jnp.float32),
            pltpu.VMEM_SHARED((HOT, D), jnp.float32),
            pltpu.SemaphoreType.DMA,
            pltpu.SemaphoreType.DMA,
            pltpu.SemaphoreType.DMA,
            pltpu.SemaphoreType.DMA,
            pltpu.SemaphoreType.DMA,
            pltpu.SemaphoreType.DMA,
            pltpu.SemaphoreType.DMA,
        ],
    )


def _tc_loss_body(nsc_ref, psc_ref, out_ref):
    ns = nsc_ref[...]
    ps = psc_ref[...]
    def logsig(x):
        return jnp.minimum(x, 0.0) - jnp.log1p(jnp.exp(-jnp.abs(x)))
    total = jnp.sum(logsig(-ns)) + jnp.sum(logsig(ps))
    out_ref[...] = jnp.reshape(-total / np.float32(N), (1, 1))


def kernel(embed, pos_neighbors, ctx_weight):
    pos = pos_neighbors.reshape(NW, TPW)
    nsc, psc = _make_sc_scores()(
        embed, ctx_weight, jnp.asarray(_HOT_LIST), jnp.asarray(_COLD_LIST),
        jnp.asarray(_RLOC), pos)
    loss = pl.pallas_call(
        _tc_loss_body,
        out_shape=jax.ShapeDtypeStruct((1, 1), jnp.float32),
    )(nsc, psc.reshape(N, 1))
    return loss.reshape(())
